# Initial kernel scaffold; baseline (speedup 1.0000x reference)
import jax
import jax.numpy as jnp
from jax.experimental import pallas as pl


def kernel(x, R, Theta):
    raise NotImplementedError("write your pallas kernel here")


# speedup vs baseline: 1.0032x; 1.0032x over previous
import functools

import jax
import jax.numpy as jnp
from jax import lax
from jax.experimental import pallas as pl
from jax.experimental.pallas import tpu as pltpu
from jax.experimental.pallas import tpu_sc as plsc

_B = 16384
_NW = 32
_BPW = _B // _NW
_GCH = 128
_NCHUNK = _BPW // _GCH
_N = 1000000


def _tc_sum_body(t_ref, o_ref):
    o_ref[...] = jnp.full((1, 16), jnp.sum(t_ref[...]), jnp.float32)


def _sc_body(xi, xj, xk, r_hbm, t_hbm, s_hbm, o_hbm,
             ii, ij, ik, ri, rj, tk, po, sv, sem_i, sem_j, sem_k):
    wid = lax.axis_index("s") * 2 + lax.axis_index("c")
    pltpu.sync_copy(xi.at[wid], ii)
    pltpu.sync_copy(xj.at[wid], ij)
    pltpu.sync_copy(xk.at[wid], ik)
    pltpu.sync_copy(s_hbm, sv)
    cps = []
    for c in range(_NCHUNK):
        dst = pl.ds(c * _GCH, _GCH)
        cps.append(pltpu.async_copy(r_hbm.at[ii.at[c]], ri.at[dst], sem_i))
        cps.append(pltpu.async_copy(r_hbm.at[ij.at[c]], rj.at[dst], sem_j))
        cps.append(pltpu.async_copy(t_hbm.at[ik.at[c]], tk.at[dst], sem_k))
    for cp in cps:
        cp.wait()
    s = sv[...]
    for t in range(_BPW // 16):
        sl = pl.ds(t * 16, 16)
        po[sl] = tk[sl] / s * (ri[sl] - rj[sl])
    pltpu.sync_copy(po, o_hbm.at[pl.ds(wid * _BPW, _BPW)])


@jax.jit
def kernel(x, R, Theta):
    xt = x.T
    xi = xt[0].reshape(_NW, _NCHUNK, _GCH)
    xj = xt[1].reshape(_NW, _NCHUNK, _GCH)
    xk = xt[2].reshape(_NW, _NCHUNK, _GCH)

    s16 = pl.pallas_call(
        _tc_sum_body,
        out_shape=jax.ShapeDtypeStruct((1, 16), jnp.float32),
    )(Theta.reshape(15625, 64))

    mesh = plsc.VectorSubcoreMesh(core_axis_name="c", subcore_axis_name="s")
    sc = functools.partial(
        pl.kernel,
        mesh=mesh,
        out_type=jax.ShapeDtypeStruct((_B,), jnp.float32),
        scratch_types=[
            pltpu.VMEM((_NCHUNK, _GCH), jnp.int32),
            pltpu.VMEM((_NCHUNK, _GCH), jnp.int32),
            pltpu.VMEM((_NCHUNK, _GCH), jnp.int32),
            pltpu.VMEM((_BPW,), jnp.float32),
            pltpu.VMEM((_BPW,), jnp.float32),
            pltpu.VMEM((_BPW,), jnp.float32),
            pltpu.VMEM((_BPW,), jnp.float32),
            pltpu.VMEM((16,), jnp.float32),
            pltpu.SemaphoreType.DMA,
            pltpu.SemaphoreType.DMA,
            pltpu.SemaphoreType.DMA,
        ],
    )(_sc_body)
    p = sc(xi, xj, xk, R.reshape(-1), Theta.reshape(-1), s16.reshape(16))
    return p.reshape(_B, 1)


# speedup vs baseline: 1.0602x; 1.0569x over previous
import functools

import jax
import jax.numpy as jnp
from jax import lax
from jax.experimental import pallas as pl
from jax.experimental.pallas import tpu as pltpu
from jax.experimental.pallas import tpu_sc as plsc

_B = 16384
_NW = 32
_BPW = _B // _NW
_GCH = 128
_NCH = _BPW // _GCH
_N = 1000000
_CH = 62496
_TAIL = _N - 16 * _CH
_UN = 18
_OUTER = _CH // 16 // _UN
_NACC = 6


def _sc_body(xi, xj, xk, r_hbm, t_hbm, o_hbm, px_hbm,
             ii, ij, ik, rg, jg, tg, sb, tb, pp, po, shv,
             sem_t, sem_i, sem_j, sem_k):
    cid = lax.axis_index("c")
    sid = lax.axis_index("s")
    wid = sid * 2 + cid
    cp_t = pltpu.async_copy(t_hbm.at[pl.ds(sid * _CH, _CH)], sb, sem_t)
    pltpu.sync_copy(xi.at[wid], ii)
    pltpu.sync_copy(xj.at[wid], ij)
    pltpu.sync_copy(xk.at[wid], ik)
    cps = []
    for c in range(_NCH):
        d = pl.ds(c * _GCH, _GCH)
        cps.append(pltpu.async_copy(r_hbm.at[ii.at[c]], rg.at[d], sem_i))
        cps.append(pltpu.async_copy(r_hbm.at[ij.at[c]], jg.at[d], sem_j))
        cps.append(pltpu.async_copy(t_hbm.at[ik.at[c]], tg.at[d], sem_k))
    cp_t.wait()
    iota = lax.iota(jnp.int32, 16)

    def sbody(t, accs):
        base = t * (_UN * 16)
        accs = list(accs)
        for k in range(_UN):
            v = sb[pl.ds(base + k * 16, 16)]
            accs[k % _NACC] = accs[k % _NACC] + v
        return tuple(accs)

    zf = jnp.zeros((16,), jnp.float32)
    accs = lax.fori_loop(0, _OUTER, sbody, (zf,) * _NACC)
    acc = accs[0]
    for a in accs[1:]:
        acc = acc + a
    pp[...] = acc

    @pl.when(sid == 15)
    def _():
        pltpu.sync_copy(t_hbm.at[pl.ds(16 * _CH, _TAIL)], tb)
        a = pp[...]
        for k in range(_TAIL // 16):
            a = a + tb[pl.ds(k * 16, 16)]
        pp[...] = a

    pltpu.sync_copy(pp, px_hbm.at[cid, sid])
    plsc.subcore_barrier()
    pltpu.sync_copy(px_hbm.at[cid], shv)
    v = shv[0]
    for r in range(1, 16):
        v = v + shv[r]
    tot = v[0]
    for l in range(1, 16):
        tot = tot + v[l]
    inv = jnp.float32(1.0) / jnp.full((16,), tot, jnp.float32)

    for cp in cps:
        cp.wait()
    for t in range(_BPW // 16):
        sl = pl.ds(t * 16, 16)
        po[sl] = tg[sl] * inv * (rg[sl] - jg[sl])
    pltpu.sync_copy(po, o_hbm.at[pl.ds(wid * _BPW, _BPW)])


@jax.jit
def kernel(x, R, Theta):
    xt = x.T
    xi = xt[0].reshape(_NW, _NCH, _GCH)
    xj = xt[1].reshape(_NW, _NCH, _GCH)
    xk = xt[2].reshape(_NW, _NCH, _GCH)
    rflat = R.T.reshape(_N)
    tflat = Theta.T.reshape(_N)
    mesh = plsc.VectorSubcoreMesh(core_axis_name="c", subcore_axis_name="s")
    sc = functools.partial(
        pl.kernel,
        mesh=mesh,
        out_type=(jax.ShapeDtypeStruct((_B,), jnp.float32),
                  jax.ShapeDtypeStruct((2, 16, 16), jnp.float32)),
        scratch_types=[
            pltpu.VMEM((_NCH, _GCH), jnp.int32),
            pltpu.VMEM((_NCH, _GCH), jnp.int32),
            pltpu.VMEM((_NCH, _GCH), jnp.int32),
            pltpu.VMEM((_BPW,), jnp.float32),
            pltpu.VMEM((_BPW,), jnp.float32),
            pltpu.VMEM((_BPW,), jnp.float32),
            pltpu.VMEM((_CH,), jnp.float32),
            pltpu.VMEM((_TAIL,), jnp.float32),
            pltpu.VMEM((16,), jnp.float32),
            pltpu.VMEM((_BPW,), jnp.float32),
            pltpu.VMEM((16, 16), jnp.float32),
            pltpu.SemaphoreType.DMA,
            pltpu.SemaphoreType.DMA,
            pltpu.SemaphoreType.DMA,
            pltpu.SemaphoreType.DMA,
        ],
    )(_sc_body)
    p, _ = sc(xi, xj, xk, rflat, tflat)
    return p.reshape(_B, 1)


# speedup vs baseline: 2.8525x; 2.6905x over previous
import functools

import jax
import jax.numpy as jnp
from jax import lax
from jax.experimental import pallas as pl
from jax.experimental.pallas import tpu as pltpu
from jax.experimental.pallas import tpu_sc as plsc

_B = 16384
_NW = 32
_BPW = _B // _NW
_GCH = 128
_NCH = _BPW // _GCH
_N = 1000000
_NPAD = 1000448
_CH = _NPAD // 16
_UN = 18
_OUTER = 217
_REM = _CH // 16 - _OUTER * _UN
_NACC = 6


def _sc_body(xi, xj, xk, r_hbm, t_hbm, o_hbm, px_hbm,
             ii, ij, ik, rg, jg, tg, sb, pp, po, shv,
             sem_t, sem_i, sem_j, sem_k):
    cid = lax.axis_index("c")
    sid = lax.axis_index("s")
    wid = sid * 2 + cid
    cp_t = pltpu.async_copy(t_hbm.at[pl.ds(sid * _CH, _CH)], sb, sem_t)
    pltpu.sync_copy(xi.at[wid], ii)
    pltpu.sync_copy(xj.at[wid], ij)
    pltpu.sync_copy(xk.at[wid], ik)
    cps = []
    for c in range(_NCH):
        d = pl.ds(c * _GCH, _GCH)
        cps.append(pltpu.async_copy(r_hbm.at[ii.at[c]], rg.at[d], sem_i))
        cps.append(pltpu.async_copy(r_hbm.at[ij.at[c]], jg.at[d], sem_j))
        cps.append(pltpu.async_copy(t_hbm.at[ik.at[c]], tg.at[d], sem_k))
    cp_t.wait()

    def sbody(t, accs):
        base = t * (_UN * 16)
        accs = list(accs)
        for k in range(_UN):
            v = sb[pl.ds(base + k * 16, 16)]
            accs[k % _NACC] = accs[k % _NACC] + v
        return tuple(accs)

    zf = jnp.zeros((16,), jnp.float32)
    accs = lax.fori_loop(0, _OUTER, sbody, (zf,) * _NACC)
    acc = accs[0]
    for a in accs[1:]:
        acc = acc + a
    for k in range(_REM):
        acc = acc + sb[pl.ds((_OUTER * _UN + k) * 16, 16)]
    pp[...] = acc

    pltpu.sync_copy(pp, px_hbm.at[cid, sid])
    plsc.subcore_barrier()
    pltpu.sync_copy(px_hbm.at[cid], shv)
    v = shv[0]
    for r in range(1, 16):
        v = v + shv[r]
    tot = v[0]
    for l in range(1, 16):
        tot = tot + v[l]
    inv = jnp.float32(1.0) / jnp.full((16,), tot, jnp.float32)

    for cp in cps:
        cp.wait()
    for t in range(_BPW // 16):
        sl = pl.ds(t * 16, 16)
        po[sl] = tg[sl] * inv * (rg[sl] - jg[sl])
    pltpu.sync_copy(po, o_hbm.at[pl.ds(wid * _BPW, _BPW)])


@jax.jit
def kernel(x, R, Theta):
    xt = x.T
    xi = xt[0].reshape(_NW, _NCH, _GCH)
    xj = xt[1].reshape(_NW, _NCH, _GCH)
    xk = xt[2].reshape(_NW, _NCH, _GCH)
    rflat = jnp.pad(R, ((0, _NPAD - _N), (0, 0))).reshape(_NPAD)
    tflat = jnp.pad(Theta, ((0, _NPAD - _N), (0, 0))).reshape(_NPAD)
    mesh = plsc.VectorSubcoreMesh(core_axis_name="c", subcore_axis_name="s")
    sc = functools.partial(
        pl.kernel,
        mesh=mesh,
        out_type=(jax.ShapeDtypeStruct((_B,), jnp.float32),
                  jax.ShapeDtypeStruct((2, 16, 16), jnp.float32)),
        scratch_types=[
            pltpu.VMEM((_NCH, _GCH), jnp.int32),
            pltpu.VMEM((_NCH, _GCH), jnp.int32),
            pltpu.VMEM((_NCH, _GCH), jnp.int32),
            pltpu.VMEM((_BPW,), jnp.float32),
            pltpu.VMEM((_BPW,), jnp.float32),
            pltpu.VMEM((_BPW,), jnp.float32),
            pltpu.VMEM((_CH,), jnp.float32),
            pltpu.VMEM((16,), jnp.float32),
            pltpu.VMEM((_BPW,), jnp.float32),
            pltpu.VMEM((16, 16), jnp.float32),
            pltpu.SemaphoreType.DMA,
            pltpu.SemaphoreType.DMA,
            pltpu.SemaphoreType.DMA,
            pltpu.SemaphoreType.DMA,
        ],
    )(_sc_body)
    p, _ = sc(xi, xj, xk, rflat, tflat)
    return p.reshape(_B, 1)
